# Initial kernel scaffold; baseline (speedup 1.0000x reference)
import jax
import jax.numpy as jnp
from jax.experimental import pallas as pl


def kernel(x_transaction, edge_index, edge_type, W_in, b_in, emb_account, emb_merchant, W_rel0, W_root0, b0, g0, beta0, W_rel1, W_root1, b1, g1, beta1, Wc1, bc1, gc, betac, Wc2, bc2):
    raise NotImplementedError("write your pallas kernel here")


# speedup vs baseline: 9.5648x; 9.5648x over previous
import functools

import jax
import jax.numpy as jnp
from jax import lax
from jax.experimental import pallas as pl
from jax.experimental.pallas import tpu as pltpu
from jax.experimental.pallas import tpu_sc as plsc

N_T_, N_A_, N_M_ = 50000, 5000, 5000
NN = N_T_ + N_A_ + N_M_
EE = 600000
DIN, HH, NREL = 128, 64, 7

E_PAD = 602112
CH = 128
NSUB = 16
NCORE = 2

ACC_D = 241664
DEG_SPAN = 15104
DEG_SUB = 944
WINV_SZ = 2 * ACC_D
WIDX_SHIFT = ACC_D - 4 * NN
TRASH_W = ACC_D + 3 * NN

NHALF = NN // 2
ACC_ROWS = NHALF + 16


def _nan_fix(x, pos, neg):
    y = jnp.where(jnp.isnan(x), 0.0, x)
    y = jnp.where(x == jnp.inf, pos, y)
    y = jnp.where(x == -jnp.inf, neg, y)
    return y


def _proj_body(x_ref, w_ref, b_ref, o_ref):
    xb = jnp.clip(x_ref[...], -10.0, 10.0)
    o_ref[...] = jnp.dot(xb, w_ref[...], preferred_element_type=jnp.float32) + b_ref[...]


def _input_proj(xt, w, b):
    blk = 2000
    return pl.pallas_call(
        _proj_body,
        grid=(N_T_ // blk,),
        in_specs=[
            pl.BlockSpec((blk, DIN), lambda i: (i, 0)),
            pl.BlockSpec((DIN, HH), lambda i: (0, 0)),
            pl.BlockSpec((1, HH), lambda i: (0, 0)),
        ],
        out_specs=pl.BlockSpec((blk, HH), lambda i: (i, 0)),
        out_shape=jax.ShapeDtypeStruct((N_T_, HH), jnp.float32),
    )(xt, w, b.reshape(1, HH))


def _transforms_body(x_ref, wrel_ref, wroot_ref, b_ref, y_ref, base_ref):
    xb = x_ref[...]
    for r in range(NREL):
        y_ref[r] = jnp.dot(xb, wrel_ref[r], preferred_element_type=jnp.float32)
    base_ref[...] = jnp.dot(xb, wroot_ref[...], preferred_element_type=jnp.float32) + b_ref[...]


def _transforms(x, w_rel, w_root, b):
    blk = 2000
    y, base = pl.pallas_call(
        _transforms_body,
        grid=(NN // blk,),
        in_specs=[
            pl.BlockSpec((blk, HH), lambda i: (i, 0)),
            pl.BlockSpec((NREL, HH, HH), lambda i: (0, 0, 0)),
            pl.BlockSpec((HH, HH), lambda i: (0, 0)),
            pl.BlockSpec((1, HH), lambda i: (0, 0)),
        ],
        out_specs=[
            pl.BlockSpec((NREL, blk, HH), lambda i: (0, i, 0)),
            pl.BlockSpec((blk, HH), lambda i: (i, 0)),
        ],
        out_shape=[
            jax.ShapeDtypeStruct((NREL, NN, HH), jnp.float32),
            jax.ShapeDtypeStruct((NN, HH), jnp.float32),
        ],
    )(x, w_rel, w_root, b.reshape(1, HH))
    return y.reshape(NREL * NN, HH), base


def _post_body(x_ref, g_ref, b_ref, o_ref):
    x = x_ref[...]
    mu = jnp.mean(x, axis=1, keepdims=True)
    var = jnp.mean((x - mu) ** 2, axis=1, keepdims=True)
    y = (x - mu) / jnp.sqrt(var + 1e-5) * g_ref[...] + b_ref[...]
    y = jnp.maximum(y, 0.0)
    o_ref[...] = _nan_fix(y, 1.0, -1.0)


def _post(x, g, beta):
    blk = 2000
    return pl.pallas_call(
        _post_body,
        grid=(NN // blk,),
        in_specs=[
            pl.BlockSpec((blk, HH), lambda i: (i, 0)),
            pl.BlockSpec((1, HH), lambda i: (0, 0)),
            pl.BlockSpec((1, HH), lambda i: (0, 0)),
        ],
        out_specs=pl.BlockSpec((blk, HH), lambda i: (i, 0)),
        out_shape=jax.ShapeDtypeStruct((NN, HH), jnp.float32),
    )(x, g.reshape(1, HH), beta.reshape(1, HH))


def _clf_body(t_ref, w1_ref, b1_ref, g_ref, bt_ref, w2_ref, b2_ref, o_ref):
    h = jnp.dot(t_ref[...], w1_ref[...], preferred_element_type=jnp.float32) + b1_ref[...]
    mu = jnp.mean(h, axis=1, keepdims=True)
    var = jnp.mean((h - mu) ** 2, axis=1, keepdims=True)
    h = (h - mu) / jnp.sqrt(var + 1e-5) * g_ref[...] + bt_ref[...]
    h = jnp.maximum(h, 0.0)
    lo = jnp.dot(h, w2_ref[...], preferred_element_type=jnp.float32) + b2_ref[...]
    o_ref[...] = _nan_fix(lo, 5.0, -5.0)


def _classifier(t, w1, b1, g, beta, w2, b2):
    blk = 2000
    hh2 = HH // 2
    return pl.pallas_call(
        _clf_body,
        grid=(N_T_ // blk,),
        in_specs=[
            pl.BlockSpec((blk, HH), lambda i: (i, 0)),
            pl.BlockSpec((HH, hh2), lambda i: (0, 0)),
            pl.BlockSpec((1, hh2), lambda i: (0, 0)),
            pl.BlockSpec((1, hh2), lambda i: (0, 0)),
            pl.BlockSpec((1, hh2), lambda i: (0, 0)),
            pl.BlockSpec((hh2, 1), lambda i: (0, 0)),
            pl.BlockSpec((1, 1), lambda i: (0, 0)),
        ],
        out_specs=pl.BlockSpec((blk, 1), lambda i: (i, 0)),
        out_shape=jax.ShapeDtypeStruct((N_T_, 1), jnp.float32),
    )(t, w1, b1.reshape(1, hh2), g.reshape(1, hh2), beta.reshape(1, hh2),
      w2, b2.reshape(1, 1))


_MESH = plsc.VectorSubcoreMesh(core_axis_name="c", subcore_axis_name="s")
_SC_PARAMS = pltpu.CompilerParams(use_tc_tiling_on_sc=False)
_IOTA16 = lambda: lax.iota(jnp.int32, 16)


@functools.partial(
    pl.kernel, mesh=_MESH, compiler_params=_SC_PARAMS,
    out_type=jax.ShapeDtypeStruct((WINV_SZ,), jnp.float32),
    scratch_types=[
        pltpu.VMEM_SHARED((ACC_D,), jnp.float32),
        pltpu.VMEM((DEG_SUB,), jnp.float32),
        pltpu.VMEM((CH,), jnp.int32),
        pltpu.VMEM((CH,), jnp.int32),
        pltpu.VMEM((CH,), jnp.int32),
        pltpu.VMEM((CH,), jnp.float32),
        pltpu.VMEM((16,), jnp.float32),
    ],
)
def _sc_deg(dst_hbm, et_hbm, winv_hbm, acc, tbuf, dstc, etc_, idxr, onesb, z16):
    c = lax.axis_index("c")
    s = lax.axis_index("s")
    nrel = 4 - c
    rbase = 4 * c
    trash = 60000 * nrel

    for j in range(DEG_SUB // 16):
        tbuf[pl.ds(j * 16, 16)] = jnp.zeros((16,), jnp.float32)
    for j in range(CH // 16):
        onesb[pl.ds(j * 16, 16)] = jnp.ones((16,), jnp.float32)
    z16[...] = jnp.zeros((16,), jnp.float32)
    for i in range(NSUB):
        pltpu.sync_copy(tbuf, acc.at[pl.ds(s * DEG_SPAN + i * DEG_SUB, DEG_SUB)])
    plsc.subcore_barrier()

    span = E_PAD // NSUB
    nch = span // CH

    def hist_chunk(i, _):
        ebase = s * span + i * CH
        pltpu.sync_copy(dst_hbm.at[pl.ds(ebase, CH)], dstc)
        pltpu.sync_copy(et_hbm.at[pl.ds(ebase, CH)], etc_)
        for k in range(CH // 16):
            dv = dstc[pl.ds(k * 16, 16)]
            ev = etc_[pl.ds(k * 16, 16)]
            rel = ev - rbase
            own = (rel >= 0) & (rel < nrel) & (dv >= 0) & (dv < NN)
            hx = jnp.where(own, rel * NN + dv, trash + _IOTA16())
            idxr[pl.ds(k * 16, 16)] = hx
        pltpu.sync_copy(onesb, acc.at[idxr], add=True)
        return ()

    lax.fori_loop(0, nch, hist_chunk, ())
    plsc.subcore_barrier()

    def winv_chunk(i, _):
        off = s * DEG_SPAN + i * DEG_SUB
        pltpu.sync_copy(acc.at[pl.ds(off, DEG_SUB)], tbuf)
        for j in range(DEG_SUB // 16):
            v = tbuf[pl.ds(j * 16, 16)]
            tbuf[pl.ds(j * 16, 16)] = 1.0 / jnp.maximum(v, 1.0)
        pltpu.sync_copy(tbuf, winv_hbm.at[pl.ds(c * ACC_D + off, DEG_SUB)])
        return ()

    lax.fori_loop(0, NSUB, winv_chunk, ())
    plsc.subcore_barrier()

    @pl.when((c == 1) & (s == 0))
    def _():
        pltpu.sync_copy(z16, winv_hbm.at[pl.ds(TRASH_W, 16)])


@functools.partial(
    pl.kernel, mesh=_MESH, compiler_params=_SC_PARAMS,
    out_type=[
        jax.ShapeDtypeStruct((E_PAD,), jnp.float32),
        jax.ShapeDtypeStruct((E_PAD,), jnp.int32),
    ],
    scratch_types=[
        pltpu.VMEM((CH,), jnp.int32),
        pltpu.VMEM((CH,), jnp.int32),
        pltpu.VMEM((CH,), jnp.int32),
        pltpu.VMEM((CH,), jnp.int32),
        pltpu.VMEM((CH,), jnp.int32),
        pltpu.VMEM((CH,), jnp.float32),
        pltpu.SemaphoreType.DMA,
    ],
)
def _sc_wsweep(src_hbm, dst_hbm, et_hbm, winv_hbm, w_hbm, yidx_hbm,
               srcc, dstc, etc_, idxr, ybuf, wbuf, sem):
    c = lax.axis_index("c")
    s = lax.axis_index("s")
    wid = s * NCORE + c
    span = E_PAD // (NSUB * NCORE)
    nch = span // CH

    def chunk(i, _):
        ebase = wid * span + i * CH
        pltpu.sync_copy(src_hbm.at[pl.ds(ebase, CH)], srcc)
        pltpu.sync_copy(dst_hbm.at[pl.ds(ebase, CH)], dstc)
        pltpu.sync_copy(et_hbm.at[pl.ds(ebase, CH)], etc_)
        for k in range(CH // 16):
            sv = srcc[pl.ds(k * 16, 16)]
            dv = dstc[pl.ds(k * 16, 16)]
            ev = etc_[pl.ds(k * 16, 16)]
            g = ev * NN + dv + jnp.where(ev >= 4, WIDX_SHIFT, 0)
            idxr[pl.ds(k * 16, 16)] = jnp.minimum(g, TRASH_W)
            ybuf[pl.ds(k * 16, 16)] = ev * NN + sv
        pltpu.async_copy(winv_hbm.at[idxr], wbuf, sem).wait()
        pltpu.sync_copy(wbuf, w_hbm.at[pl.ds(ebase, CH)])
        pltpu.sync_copy(ybuf, yidx_hbm.at[pl.ds(ebase, CH)])
        return ()

    lax.fori_loop(0, nch, chunk, ())


@functools.partial(
    pl.kernel, mesh=_MESH, compiler_params=_SC_PARAMS,
    out_type=jax.ShapeDtypeStruct((NN, HH), jnp.float32),
    scratch_types=[
        pltpu.VMEM_SHARED((ACC_ROWS, HH), jnp.float32),
        pltpu.VMEM((CH,), jnp.int32),
        pltpu.VMEM((CH,), jnp.int32),
        pltpu.VMEM((CH,), jnp.float32),
        pltpu.VMEM((CH,), jnp.int32),
        pltpu.VMEM((CH, HH), jnp.float32),
        pltpu.VMEM((16, HH), jnp.float32),
        pltpu.SemaphoreType.DMA,
    ],
)
def _sc_edge(yflat_hbm, base_hbm, yidx_hbm, dst_hbm, w_hbm, out_hbm,
             acc, yic, dstc, wc, ldst, rowbuf, z16, sem):
    c = lax.axis_index("c")
    s = lax.axis_index("s")
    lo = c * NHALF

    rows_pt = 1880
    last = NSUB - 1

    @pl.when(s < last)
    def _():
        pltpu.sync_copy(base_hbm.at[pl.ds(lo + s * rows_pt, rows_pt)],
                        acc.at[pl.ds(s * rows_pt, rows_pt)])

    @pl.when(s == last)
    def _():
        rem = NHALF - last * rows_pt
        pltpu.sync_copy(base_hbm.at[pl.ds(lo + last * rows_pt, rem)],
                        acc.at[pl.ds(last * rows_pt, rem)])
        for j in range(16):
            for q in range(HH // 16):
                z16[j, pl.ds(q * 16, 16)] = jnp.zeros((16,), jnp.float32)
        pltpu.sync_copy(z16, acc.at[pl.ds(NHALF, 16)])

    plsc.subcore_barrier()

    span = E_PAD // NSUB
    nch = span // CH

    def chunk(i, _):
        ebase = s * span + i * CH
        pltpu.sync_copy(yidx_hbm.at[pl.ds(ebase, CH)], yic)
        pltpu.sync_copy(dst_hbm.at[pl.ds(ebase, CH)], dstc)
        pltpu.sync_copy(w_hbm.at[pl.ds(ebase, CH)], wc)
        pltpu.async_copy(yflat_hbm.at[yic], rowbuf, sem).wait()
        for k in range(CH // 16):
            dv = dstc[pl.ds(k * 16, 16)]
            l = dv - lo
            own = (l >= 0) & (l < NHALF)
            ldst[pl.ds(k * 16, 16)] = jnp.where(own, l, NHALF + _IOTA16())

        for k in range(CH // 16):
            wv = wc[pl.ds(k * 16, 16)]
            for lane in range(16):
                we = wv[lane]
                e = k * 16 + lane
                for q in range(HH // 16):
                    rowbuf[e, pl.ds(q * 16, 16)] = rowbuf[e, pl.ds(q * 16, 16)] * jnp.full((16,), we)
        pltpu.sync_copy(rowbuf, acc.at[ldst], add=True)
        return ()

    lax.fori_loop(0, nch, chunk, ())
    plsc.subcore_barrier()

    @pl.when(s < last)
    def _():
        pltpu.sync_copy(acc.at[pl.ds(s * rows_pt, rows_pt)],
                        out_hbm.at[pl.ds(lo + s * rows_pt, rows_pt)])

    @pl.when(s == last)
    def _():
        rem = NHALF - last * rows_pt
        pltpu.sync_copy(acc.at[pl.ds(last * rows_pt, rem)],
                        out_hbm.at[pl.ds(lo + last * rows_pt, rem)])


def kernel(x_transaction, edge_index, edge_type, W_in, b_in, emb_account,
           emb_merchant, W_rel0, W_root0, b0, g0, beta0, W_rel1, W_root1, b1,
           g1, beta1, Wc1, bc1, gc, betac, Wc2, bc2):
    xt = _input_proj(x_transaction, W_in, b_in)
    x = jnp.concatenate([xt, emb_account, emb_merchant], axis=0)

    pad = E_PAD - EE
    src = jnp.concatenate([edge_index[0], jnp.zeros((pad,), jnp.int32)])
    dst = jnp.concatenate([edge_index[1], jnp.full((pad,), 1 << 29, jnp.int32)])
    et = jnp.concatenate([edge_type, jnp.zeros((pad,), jnp.int32)])

    winv = _sc_deg(dst, et)
    w, yidx = _sc_wsweep(src, dst, et, winv)

    y0, base0 = _transforms(x, W_rel0, W_root0, b0)
    x = _post(_sc_edge(y0, base0, yidx, dst, w), g0, beta0)
    y1, base1 = _transforms(x, W_rel1, W_root1, b1)
    x = _post(_sc_edge(y1, base1, yidx, dst, w), g1, beta1)

    return _classifier(x[:N_T_], Wc1, bc1, gc, betac, Wc2, bc2)


# speedup vs baseline: 12.0532x; 1.2602x over previous
import functools

import jax
import jax.numpy as jnp
from jax import lax
from jax.experimental import pallas as pl
from jax.experimental.pallas import tpu as pltpu
from jax.experimental.pallas import tpu_sc as plsc

N_T_, N_A_, N_M_ = 50000, 5000, 5000
NN = N_T_ + N_A_ + N_M_
EE = 600000
DIN, HH, NREL = 128, 64, 7

E_PAD = 602112
CH = 128
NSUB = 16
NCORE = 2

ACC_D = 241664
DEG_SPAN = 15104
DEG_SUB = 944
WINV_SZ = 2 * ACC_D
WIDX_SHIFT = ACC_D - 4 * NN
TRASH_W = ACC_D + 3 * NN

NHALF = NN // 2
ACC_ROWS = NHALF + 8


def _nan_fix(x, pos, neg):
    y = jnp.where(jnp.isnan(x), 0.0, x)
    y = jnp.where(x == jnp.inf, pos, y)
    y = jnp.where(x == -jnp.inf, neg, y)
    return y


def _proj_body(x_ref, w_ref, b_ref, o_ref):
    xb = jnp.clip(x_ref[...], -10.0, 10.0)
    o_ref[...] = jnp.dot(xb, w_ref[...], preferred_element_type=jnp.float32) + b_ref[...]


def _input_proj(xt, w, b):
    blk = 2000
    return pl.pallas_call(
        _proj_body,
        grid=(N_T_ // blk,),
        in_specs=[
            pl.BlockSpec((blk, DIN), lambda i: (i, 0)),
            pl.BlockSpec((DIN, HH), lambda i: (0, 0)),
            pl.BlockSpec((1, HH), lambda i: (0, 0)),
        ],
        out_specs=pl.BlockSpec((blk, HH), lambda i: (i, 0)),
        out_shape=jax.ShapeDtypeStruct((N_T_, HH), jnp.float32),
    )(xt, w, b.reshape(1, HH))


def _transforms_body(x_ref, wrel_ref, wroot_ref, b_ref, y_ref, base_ref):
    xb = x_ref[...]
    for r in range(NREL):
        y_ref[r] = jnp.dot(xb, wrel_ref[r], preferred_element_type=jnp.float32)
    base_ref[...] = jnp.dot(xb, wroot_ref[...], preferred_element_type=jnp.float32) + b_ref[...]


def _transforms(x, w_rel, w_root, b):
    blk = 2000
    y, base = pl.pallas_call(
        _transforms_body,
        grid=(NN // blk,),
        in_specs=[
            pl.BlockSpec((blk, HH), lambda i: (i, 0)),
            pl.BlockSpec((NREL, HH, HH), lambda i: (0, 0, 0)),
            pl.BlockSpec((HH, HH), lambda i: (0, 0)),
            pl.BlockSpec((1, HH), lambda i: (0, 0)),
        ],
        out_specs=[
            pl.BlockSpec((NREL, blk, HH), lambda i: (0, i, 0)),
            pl.BlockSpec((blk, HH), lambda i: (i, 0)),
        ],
        out_shape=[
            jax.ShapeDtypeStruct((NREL, NN, HH), jnp.float32),
            jax.ShapeDtypeStruct((NN, HH), jnp.float32),
        ],
    )(x, w_rel, w_root, b.reshape(1, HH))
    return y.reshape(NREL * NN, HH), base


def _post_body(x_ref, g_ref, b_ref, o_ref):
    x = x_ref[...]
    mu = jnp.mean(x, axis=1, keepdims=True)
    var = jnp.mean((x - mu) ** 2, axis=1, keepdims=True)
    y = (x - mu) / jnp.sqrt(var + 1e-5) * g_ref[...] + b_ref[...]
    y = jnp.maximum(y, 0.0)
    o_ref[...] = _nan_fix(y, 1.0, -1.0)


def _post(x, g, beta):
    blk = 2000
    return pl.pallas_call(
        _post_body,
        grid=(NN // blk,),
        in_specs=[
            pl.BlockSpec((blk, HH), lambda i: (i, 0)),
            pl.BlockSpec((1, HH), lambda i: (0, 0)),
            pl.BlockSpec((1, HH), lambda i: (0, 0)),
        ],
        out_specs=pl.BlockSpec((blk, HH), lambda i: (i, 0)),
        out_shape=jax.ShapeDtypeStruct((NN, HH), jnp.float32),
    )(x, g.reshape(1, HH), beta.reshape(1, HH))


def _clf_body(t_ref, w1_ref, b1_ref, g_ref, bt_ref, w2_ref, b2_ref, o_ref):
    h = jnp.dot(t_ref[...], w1_ref[...], preferred_element_type=jnp.float32) + b1_ref[...]
    mu = jnp.mean(h, axis=1, keepdims=True)
    var = jnp.mean((h - mu) ** 2, axis=1, keepdims=True)
    h = (h - mu) / jnp.sqrt(var + 1e-5) * g_ref[...] + bt_ref[...]
    h = jnp.maximum(h, 0.0)
    lo = jnp.dot(h, w2_ref[...], preferred_element_type=jnp.float32) + b2_ref[...]
    o_ref[...] = _nan_fix(lo, 5.0, -5.0)


def _classifier(t, w1, b1, g, beta, w2, b2):
    blk = 2000
    hh2 = HH // 2
    return pl.pallas_call(
        _clf_body,
        grid=(N_T_ // blk,),
        in_specs=[
            pl.BlockSpec((blk, HH), lambda i: (i, 0)),
            pl.BlockSpec((HH, hh2), lambda i: (0, 0)),
            pl.BlockSpec((1, hh2), lambda i: (0, 0)),
            pl.BlockSpec((1, hh2), lambda i: (0, 0)),
            pl.BlockSpec((1, hh2), lambda i: (0, 0)),
            pl.BlockSpec((hh2, 1), lambda i: (0, 0)),
            pl.BlockSpec((1, 1), lambda i: (0, 0)),
        ],
        out_specs=pl.BlockSpec((blk, 1), lambda i: (i, 0)),
        out_shape=jax.ShapeDtypeStruct((N_T_, 1), jnp.float32),
    )(t, w1, b1.reshape(1, hh2), g.reshape(1, hh2), beta.reshape(1, hh2),
      w2, b2.reshape(1, 1))


_MESH = plsc.VectorSubcoreMesh(core_axis_name="c", subcore_axis_name="s")
_SC_PARAMS = pltpu.CompilerParams(use_tc_tiling_on_sc=False)
_IOTA16 = lambda: lax.iota(jnp.int32, 16)


@functools.partial(
    pl.kernel, mesh=_MESH, compiler_params=_SC_PARAMS,
    out_type=jax.ShapeDtypeStruct((WINV_SZ,), jnp.float32),
    scratch_types=[
        pltpu.VMEM_SHARED((ACC_D,), jnp.float32),
        pltpu.VMEM((DEG_SUB,), jnp.float32),
        pltpu.VMEM((CH,), jnp.int32),
        pltpu.VMEM((CH,), jnp.int32),
        pltpu.VMEM((CH,), jnp.int32),
        pltpu.VMEM((CH,), jnp.float32),
        pltpu.VMEM((16,), jnp.float32),
    ],
)
def _sc_deg(dst_hbm, et_hbm, winv_hbm, acc, tbuf, dstc, etc_, idxr, onesb, z16):
    c = lax.axis_index("c")
    s = lax.axis_index("s")
    nrel = 4 - c
    rbase = 4 * c
    trash = 60000 * nrel

    for j in range(DEG_SUB // 16):
        tbuf[pl.ds(j * 16, 16)] = jnp.zeros((16,), jnp.float32)
    for j in range(CH // 16):
        onesb[pl.ds(j * 16, 16)] = jnp.ones((16,), jnp.float32)
    z16[...] = jnp.zeros((16,), jnp.float32)
    for i in range(NSUB):
        pltpu.sync_copy(tbuf, acc.at[pl.ds(s * DEG_SPAN + i * DEG_SUB, DEG_SUB)])
    plsc.subcore_barrier()

    span = E_PAD // NSUB
    nch = span // CH

    def hist_chunk(i, _):
        ebase = s * span + i * CH
        pltpu.sync_copy(dst_hbm.at[pl.ds(ebase, CH)], dstc)
        pltpu.sync_copy(et_hbm.at[pl.ds(ebase, CH)], etc_)
        for k in range(CH // 16):
            dv = dstc[pl.ds(k * 16, 16)]
            ev = etc_[pl.ds(k * 16, 16)]
            rel = ev - rbase
            own = (rel >= 0) & (rel < nrel) & (dv >= 0) & (dv < NN)
            hx = jnp.where(own, rel * NN + dv, trash + _IOTA16())
            idxr[pl.ds(k * 16, 16)] = hx
        pltpu.sync_copy(onesb, acc.at[idxr], add=True)
        return ()

    lax.fori_loop(0, nch, hist_chunk, ())
    plsc.subcore_barrier()

    def winv_chunk(i, _):
        off = s * DEG_SPAN + i * DEG_SUB
        pltpu.sync_copy(acc.at[pl.ds(off, DEG_SUB)], tbuf)
        for j in range(DEG_SUB // 16):
            v = tbuf[pl.ds(j * 16, 16)]
            tbuf[pl.ds(j * 16, 16)] = 1.0 / jnp.maximum(v, 1.0)
        pltpu.sync_copy(tbuf, winv_hbm.at[pl.ds(c * ACC_D + off, DEG_SUB)])
        return ()

    lax.fori_loop(0, NSUB, winv_chunk, ())
    plsc.subcore_barrier()

    @pl.when((c == 1) & (s == 0))
    def _():
        pltpu.sync_copy(z16, winv_hbm.at[pl.ds(TRASH_W, 16)])


@functools.partial(
    pl.kernel, mesh=_MESH, compiler_params=_SC_PARAMS,
    out_type=[
        jax.ShapeDtypeStruct((E_PAD,), jnp.float32),
        jax.ShapeDtypeStruct((E_PAD,), jnp.int32),
    ],
    scratch_types=[
        pltpu.VMEM((CH,), jnp.int32),
        pltpu.VMEM((CH,), jnp.int32),
        pltpu.VMEM((CH,), jnp.int32),
        pltpu.VMEM((CH,), jnp.int32),
        pltpu.VMEM((CH,), jnp.int32),
        pltpu.VMEM((CH,), jnp.float32),
        pltpu.SemaphoreType.DMA,
    ],
)
def _sc_wsweep(src_hbm, dst_hbm, et_hbm, winv_hbm, w_hbm, yidx_hbm,
               srcc, dstc, etc_, idxr, ybuf, wbuf, sem):
    c = lax.axis_index("c")
    s = lax.axis_index("s")
    wid = s * NCORE + c
    span = E_PAD // (NSUB * NCORE)
    nch = span // CH

    def chunk(i, _):
        ebase = wid * span + i * CH
        pltpu.sync_copy(src_hbm.at[pl.ds(ebase, CH)], srcc)
        pltpu.sync_copy(dst_hbm.at[pl.ds(ebase, CH)], dstc)
        pltpu.sync_copy(et_hbm.at[pl.ds(ebase, CH)], etc_)
        for k in range(CH // 16):
            sv = srcc[pl.ds(k * 16, 16)]
            dv = dstc[pl.ds(k * 16, 16)]
            ev = etc_[pl.ds(k * 16, 16)]
            g = ev * NN + dv + jnp.where(ev >= 4, WIDX_SHIFT, 0)
            idxr[pl.ds(k * 16, 16)] = jnp.minimum(g, TRASH_W)
            ybuf[pl.ds(k * 16, 16)] = ev * NN + sv
        pltpu.async_copy(winv_hbm.at[idxr], wbuf, sem).wait()
        pltpu.sync_copy(wbuf, w_hbm.at[pl.ds(ebase, CH)])
        pltpu.sync_copy(ybuf, yidx_hbm.at[pl.ds(ebase, CH)])
        return ()

    lax.fori_loop(0, nch, chunk, ())


ECH = 64
SUP = 7
SUPE = SUP * ECH
NSUPT = (E_PAD // NSUB) // SUPE
PAIRS = NSUPT // 2


@functools.partial(
    pl.kernel, mesh=_MESH, compiler_params=_SC_PARAMS,
    out_type=jax.ShapeDtypeStruct((NN, HH), jnp.float32),
    scratch_types=[
        pltpu.VMEM_SHARED((ACC_ROWS, HH), jnp.float32),
        pltpu.VMEM((SUPE,), jnp.int32),
        pltpu.VMEM((SUPE,), jnp.int32),
        pltpu.VMEM((SUPE + 16,), jnp.float32),
        pltpu.VMEM((2, ECH), jnp.int32),
        pltpu.VMEM((2, ECH, HH), jnp.float32),
        pltpu.SemaphoreType.DMA,
        pltpu.SemaphoreType.DMA,
        pltpu.SemaphoreType.DMA,
        pltpu.SemaphoreType.DMA,
        pltpu.SemaphoreType.DMA,
        pltpu.SemaphoreType.DMA,
        pltpu.SemaphoreType.DMA,
    ],
)
def _sc_edge(yflat_hbm, base_hbm, yidx_hbm, dst_hbm, w_hbm, out_hbm,
             acc, yicS, dstS, wS, ldst2, rowbuf2,
             sem_sy, sem_sd, sem_sw, sem_g0, sem_g1, sem_w0, sem_w1):
    c = lax.axis_index("c")
    s = lax.axis_index("s")
    lo = c * NHALF
    span = E_PAD // NSUB
    sem_g = (sem_g0, sem_g1)
    sem_w = (sem_w0, sem_w1)

    rows_pt = 1880
    last = NSUB - 1

    @pl.when(s < last)
    def _():
        pltpu.sync_copy(base_hbm.at[pl.ds(lo + s * rows_pt, rows_pt)],
                        acc.at[pl.ds(s * rows_pt, rows_pt)])

    @pl.when(s == last)
    def _():
        rem = NHALF - last * rows_pt
        pltpu.sync_copy(base_hbm.at[pl.ds(lo + last * rows_pt, rem)],
                        acc.at[pl.ds(last * rows_pt, rem)])
        for j in range(8):
            for q in range(HH // 16):
                rowbuf2[0, j, pl.ds(q * 16, 16)] = jnp.zeros((16,), jnp.float32)
        pltpu.sync_copy(rowbuf2.at[0, pl.ds(0, 8)], acc.at[pl.ds(NHALF, 8)])

    plsc.subcore_barrier()

    def start_sup(u):
        off = pl.multiple_of(s * span + u * SUPE, 8)
        pltpu.async_copy(yidx_hbm.at[pl.ds(off, SUPE)], yicS, sem_sy)
        pltpu.async_copy(dst_hbm.at[pl.ds(off, SUPE)], dstS, sem_sd)
        pltpu.async_copy(w_hbm.at[pl.ds(off, SUPE)], wS.at[pl.ds(0, SUPE)],
                         sem_sw)

    def wait_sup_yic():
        pltpu.make_async_copy(yidx_hbm.at[pl.ds(0, SUPE)], yicS, sem_sy).wait()

    def wait_sup_dw():
        pltpu.make_async_copy(dst_hbm.at[pl.ds(0, SUPE)], dstS, sem_sd).wait()
        pltpu.make_async_copy(w_hbm.at[pl.ds(0, SUPE)],
                              wS.at[pl.ds(0, SUPE)], sem_sw).wait()

    def start_gather(kk, p):
        pltpu.async_copy(yflat_hbm.at[yicS.at[pl.ds(kk * ECH, ECH)]],
                         rowbuf2.at[p], sem_g[p])

    def wait_gather(p):
        pltpu.make_async_copy(yflat_hbm.at[pl.ds(0, ECH)], rowbuf2.at[p],
                              sem_g[p]).wait()

    def start_scatter(p):
        pltpu.async_copy(rowbuf2.at[p], acc.at[ldst2.at[p]], sem_w[p],
                         add=True)

    def wait_scatter(p):
        pltpu.make_async_copy(yflat_hbm.at[pl.ds(0, ECH)], rowbuf2.at[p],
                              sem_w[p]).wait()

    def compute_chunk(kk, p):
        for m in range(ECH // 16):
            dv = dstS[pl.ds(kk * ECH + m * 16, 16)]
            l = dv - lo
            own = (l >= 0) & (l < NHALF)
            ldst2[p, pl.ds(m * 16, 16)] = jnp.where(
                own, l, NHALF + jnp.bitwise_and(_IOTA16(), 7))

        def scale_body(e, _):
            wv = wS[pl.ds(kk * ECH + e, 16)]
            we = wv[0]
            for q in range(HH // 16):
                rowbuf2[p, e, pl.ds(q * 16, 16)] = (
                    rowbuf2[p, e, pl.ds(q * 16, 16)] * jnp.full((16,), we))
            return ()

        lax.fori_loop(0, ECH, scale_body, (), unroll=2)

    start_sup(0)
    wait_sup_yic()
    wait_sup_dw()
    start_gather(0, 0)

    def pair(j, _):
        for k in range(2 * SUP):
            kk = k % SUP
            p = k & 1
            q = 1 - p
            wait_gather(p)
            if k == 0:
                @pl.when(j > 0)
                def _():
                    wait_sup_dw()
            if k == SUP:
                wait_sup_dw()
            if kk != SUP - 1:
                if k == 0:
                    @pl.when(j > 0)
                    def _():
                        wait_scatter(q)
                else:
                    wait_scatter(q)
                start_gather(kk + 1, q)
                compute_chunk(kk, p)
            else:
                compute_chunk(kk, p)
                islast = (k == 2 * SUP - 1) & (j == PAIRS - 1)
                if k == 2 * SUP - 1:
                    @pl.when(j < PAIRS - 1)
                    def _():
                        start_sup(2 * j + 2)
                        wait_sup_yic()
                        wait_scatter(q)
                        start_gather(0, q)
                else:
                    start_sup(2 * j + 1)
                    wait_sup_yic()
                    wait_scatter(q)
                    start_gather(0, q)
            start_scatter(p)
        return ()

    lax.fori_loop(0, PAIRS, pair, ())
    wait_scatter(0)
    wait_scatter(1)
    plsc.subcore_barrier()

    @pl.when(s < last)
    def _():
        pltpu.sync_copy(acc.at[pl.ds(s * rows_pt, rows_pt)],
                        out_hbm.at[pl.ds(lo + s * rows_pt, rows_pt)])

    @pl.when(s == last)
    def _():
        rem = NHALF - last * rows_pt
        pltpu.sync_copy(acc.at[pl.ds(last * rows_pt, rem)],
                        out_hbm.at[pl.ds(lo + last * rows_pt, rem)])


def kernel(x_transaction, edge_index, edge_type, W_in, b_in, emb_account,
           emb_merchant, W_rel0, W_root0, b0, g0, beta0, W_rel1, W_root1, b1,
           g1, beta1, Wc1, bc1, gc, betac, Wc2, bc2):
    xt = _input_proj(x_transaction, W_in, b_in)
    x = jnp.concatenate([xt, emb_account, emb_merchant], axis=0)

    pad = E_PAD - EE
    src = jnp.concatenate([edge_index[0], jnp.zeros((pad,), jnp.int32)])
    dst = jnp.concatenate([edge_index[1], jnp.full((pad,), 1 << 29, jnp.int32)])
    et = jnp.concatenate([edge_type, jnp.zeros((pad,), jnp.int32)])

    winv = _sc_deg(dst, et)
    w, yidx = _sc_wsweep(src, dst, et, winv)

    y0, base0 = _transforms(x, W_rel0, W_root0, b0)
    x = _post(_sc_edge(y0, base0, yidx, dst, w), g0, beta0)
    y1, base1 = _transforms(x, W_rel1, W_root1, b1)
    x = _post(_sc_edge(y1, base1, yidx, dst, w), g1, beta1)

    return _classifier(x[:N_T_], Wc1, bc1, gc, betac, Wc2, bc2)


# speedup vs baseline: 12.4429x; 1.0323x over previous
import functools

import jax
import jax.numpy as jnp
from jax import lax
from jax.experimental import pallas as pl
from jax.experimental.pallas import tpu as pltpu
from jax.experimental.pallas import tpu_sc as plsc

N_T_, N_A_, N_M_ = 50000, 5000, 5000
NN = N_T_ + N_A_ + N_M_
EE = 600000
DIN, HH, NREL = 128, 64, 7

E_PAD = 602112
CH = 128
NSUB = 16
NCORE = 2

ACC_D = 241664
DEG_SPAN = 15104
DEG_SUB = 944
WINV_SZ = 2 * ACC_D
WIDX_SHIFT = ACC_D - 4 * NN
TRASH_W = ACC_D + 3 * NN

NHALF = NN // 2
ACC_ROWS = NHALF + 8


def _nan_fix(x, pos, neg):
    y = jnp.where(jnp.isnan(x), 0.0, x)
    y = jnp.where(x == jnp.inf, pos, y)
    y = jnp.where(x == -jnp.inf, neg, y)
    return y


def _proj_body(x_ref, w_ref, b_ref, o_ref):
    xb = jnp.clip(x_ref[...], -10.0, 10.0)
    o_ref[...] = jnp.dot(xb, w_ref[...], preferred_element_type=jnp.float32) + b_ref[...]


def _input_proj(xt, w, b):
    blk = 2000
    return pl.pallas_call(
        _proj_body,
        grid=(N_T_ // blk,),
        in_specs=[
            pl.BlockSpec((blk, DIN), lambda i: (i, 0)),
            pl.BlockSpec((DIN, HH), lambda i: (0, 0)),
            pl.BlockSpec((1, HH), lambda i: (0, 0)),
        ],
        out_specs=pl.BlockSpec((blk, HH), lambda i: (i, 0)),
        out_shape=jax.ShapeDtypeStruct((N_T_, HH), jnp.float32),
    )(xt, w, b.reshape(1, HH))


def _transforms_body(x_ref, wrel_ref, wroot_ref, b_ref, y_ref, base_ref):
    xb = x_ref[...]
    for r in range(NREL):
        y_ref[r] = jnp.dot(xb, wrel_ref[r], preferred_element_type=jnp.float32)
    base_ref[...] = jnp.dot(xb, wroot_ref[...], preferred_element_type=jnp.float32) + b_ref[...]


def _transforms(x, w_rel, w_root, b):
    blk = 2000
    y, base = pl.pallas_call(
        _transforms_body,
        grid=(NN // blk,),
        in_specs=[
            pl.BlockSpec((blk, HH), lambda i: (i, 0)),
            pl.BlockSpec((NREL, HH, HH), lambda i: (0, 0, 0)),
            pl.BlockSpec((HH, HH), lambda i: (0, 0)),
            pl.BlockSpec((1, HH), lambda i: (0, 0)),
        ],
        out_specs=[
            pl.BlockSpec((NREL, blk, HH), lambda i: (0, i, 0)),
            pl.BlockSpec((blk, HH), lambda i: (i, 0)),
        ],
        out_shape=[
            jax.ShapeDtypeStruct((NREL, NN, HH), jnp.float32),
            jax.ShapeDtypeStruct((NN, HH), jnp.float32),
        ],
    )(x, w_rel, w_root, b.reshape(1, HH))
    return y.reshape(NREL * NN, HH), base


def _post_body(x_ref, g_ref, b_ref, o_ref):
    x = x_ref[...]
    mu = jnp.mean(x, axis=1, keepdims=True)
    var = jnp.mean((x - mu) ** 2, axis=1, keepdims=True)
    y = (x - mu) / jnp.sqrt(var + 1e-5) * g_ref[...] + b_ref[...]
    y = jnp.maximum(y, 0.0)
    o_ref[...] = _nan_fix(y, 1.0, -1.0)


def _post(x, g, beta):
    blk = 2000
    return pl.pallas_call(
        _post_body,
        grid=(NN // blk,),
        in_specs=[
            pl.BlockSpec((blk, HH), lambda i: (i, 0)),
            pl.BlockSpec((1, HH), lambda i: (0, 0)),
            pl.BlockSpec((1, HH), lambda i: (0, 0)),
        ],
        out_specs=pl.BlockSpec((blk, HH), lambda i: (i, 0)),
        out_shape=jax.ShapeDtypeStruct((NN, HH), jnp.float32),
    )(x, g.reshape(1, HH), beta.reshape(1, HH))


def _clf_body(t_ref, w1_ref, b1_ref, g_ref, bt_ref, w2_ref, b2_ref, o_ref):
    h = jnp.dot(t_ref[...], w1_ref[...], preferred_element_type=jnp.float32) + b1_ref[...]
    mu = jnp.mean(h, axis=1, keepdims=True)
    var = jnp.mean((h - mu) ** 2, axis=1, keepdims=True)
    h = (h - mu) / jnp.sqrt(var + 1e-5) * g_ref[...] + bt_ref[...]
    h = jnp.maximum(h, 0.0)
    lo = jnp.dot(h, w2_ref[...], preferred_element_type=jnp.float32) + b2_ref[...]
    o_ref[...] = _nan_fix(lo, 5.0, -5.0)


def _classifier(t, w1, b1, g, beta, w2, b2):
    blk = 2000
    hh2 = HH // 2
    return pl.pallas_call(
        _clf_body,
        grid=(N_T_ // blk,),
        in_specs=[
            pl.BlockSpec((blk, HH), lambda i: (i, 0)),
            pl.BlockSpec((HH, hh2), lambda i: (0, 0)),
            pl.BlockSpec((1, hh2), lambda i: (0, 0)),
            pl.BlockSpec((1, hh2), lambda i: (0, 0)),
            pl.BlockSpec((1, hh2), lambda i: (0, 0)),
            pl.BlockSpec((hh2, 1), lambda i: (0, 0)),
            pl.BlockSpec((1, 1), lambda i: (0, 0)),
        ],
        out_specs=pl.BlockSpec((blk, 1), lambda i: (i, 0)),
        out_shape=jax.ShapeDtypeStruct((N_T_, 1), jnp.float32),
    )(t, w1, b1.reshape(1, hh2), g.reshape(1, hh2), beta.reshape(1, hh2),
      w2, b2.reshape(1, 1))


_MESH = plsc.VectorSubcoreMesh(core_axis_name="c", subcore_axis_name="s")
_SC_PARAMS = pltpu.CompilerParams(use_tc_tiling_on_sc=False)
_IOTA16 = lambda: lax.iota(jnp.int32, 16)


@functools.partial(
    pl.kernel, mesh=_MESH, compiler_params=_SC_PARAMS,
    out_type=jax.ShapeDtypeStruct((WINV_SZ,), jnp.float32),
    scratch_types=[
        pltpu.VMEM_SHARED((ACC_D,), jnp.float32),
        pltpu.VMEM((DEG_SUB,), jnp.float32),
        pltpu.VMEM((CH,), jnp.int32),
        pltpu.VMEM((CH,), jnp.int32),
        pltpu.VMEM((CH,), jnp.int32),
        pltpu.VMEM((CH,), jnp.float32),
        pltpu.VMEM((16,), jnp.float32),
    ],
)
def _sc_deg(dst_hbm, et_hbm, winv_hbm, acc, tbuf, dstc, etc_, idxr, onesb, z16):
    c = lax.axis_index("c")
    s = lax.axis_index("s")
    nrel = 4 - c
    rbase = 4 * c
    trash = 60000 * nrel

    for j in range(DEG_SUB // 16):
        tbuf[pl.ds(j * 16, 16)] = jnp.zeros((16,), jnp.float32)
    for j in range(CH // 16):
        onesb[pl.ds(j * 16, 16)] = jnp.ones((16,), jnp.float32)
    z16[...] = jnp.zeros((16,), jnp.float32)
    for i in range(NSUB):
        pltpu.sync_copy(tbuf, acc.at[pl.ds(s * DEG_SPAN + i * DEG_SUB, DEG_SUB)])
    plsc.subcore_barrier()

    span = E_PAD // NSUB
    nch = span // CH

    def hist_chunk(i, _):
        ebase = s * span + i * CH
        pltpu.sync_copy(dst_hbm.at[pl.ds(ebase, CH)], dstc)
        pltpu.sync_copy(et_hbm.at[pl.ds(ebase, CH)], etc_)
        for k in range(CH // 16):
            dv = dstc[pl.ds(k * 16, 16)]
            ev = etc_[pl.ds(k * 16, 16)]
            rel = ev - rbase
            own = (rel >= 0) & (rel < nrel) & (dv >= 0) & (dv < NN)
            hx = jnp.where(own, rel * NN + dv, trash + _IOTA16())
            idxr[pl.ds(k * 16, 16)] = hx
        pltpu.sync_copy(onesb, acc.at[idxr], add=True)
        return ()

    lax.fori_loop(0, nch, hist_chunk, ())
    plsc.subcore_barrier()

    def winv_chunk(i, _):
        off = s * DEG_SPAN + i * DEG_SUB
        pltpu.sync_copy(acc.at[pl.ds(off, DEG_SUB)], tbuf)
        for j in range(DEG_SUB // 16):
            v = tbuf[pl.ds(j * 16, 16)]
            tbuf[pl.ds(j * 16, 16)] = 1.0 / jnp.maximum(v, 1.0)
        pltpu.sync_copy(tbuf, winv_hbm.at[pl.ds(c * ACC_D + off, DEG_SUB)])
        return ()

    lax.fori_loop(0, NSUB, winv_chunk, ())
    plsc.subcore_barrier()

    @pl.when((c == 1) & (s == 0))
    def _():
        pltpu.sync_copy(z16, winv_hbm.at[pl.ds(TRASH_W, 16)])


WSUP = 7
WSUPE = WSUP * CH
WNSUP = (E_PAD // (NSUB * NCORE)) // WSUPE
WPAIRS = (WNSUP - 1) // 2


@functools.partial(
    pl.kernel, mesh=_MESH, compiler_params=_SC_PARAMS,
    out_type=[
        jax.ShapeDtypeStruct((E_PAD,), jnp.float32),
        jax.ShapeDtypeStruct((E_PAD,), jnp.int32),
    ],
    scratch_types=[
        pltpu.VMEM((WSUPE,), jnp.int32),
        pltpu.VMEM((WSUPE,), jnp.int32),
        pltpu.VMEM((WSUPE,), jnp.int32),
        pltpu.VMEM((2, CH), jnp.int32),
        pltpu.VMEM((2, CH), jnp.int32),
        pltpu.VMEM((2, CH), jnp.float32),
        pltpu.SemaphoreType.DMA,
        pltpu.SemaphoreType.DMA,
        pltpu.SemaphoreType.DMA,
        pltpu.SemaphoreType.DMA,
        pltpu.SemaphoreType.DMA,
        pltpu.SemaphoreType.DMA,
    ],
)
def _sc_wsweep(src_hbm, dst_hbm, et_hbm, winv_hbm, w_hbm, yidx_hbm,
               srcS, dstS, etS, idxr2, ybuf2, wbuf2,
               sem_g0, sem_g1, sem_sw0, sem_sw1, sem_sy0, sem_sy1):
    c = lax.axis_index("c")
    s = lax.axis_index("s")
    wid = s * NCORE + c
    span = E_PAD // (NSUB * NCORE)
    base = wid * span
    sem_g = (sem_g0, sem_g1)
    sem_sw = (sem_sw0, sem_sw1)
    sem_sy = (sem_sy0, sem_sy1)

    def drain_wstore(p):
        pltpu.make_async_copy(wbuf2.at[p], w_hbm.at[pl.ds(0, CH)],
                              sem_sw[p]).wait()

    def drain_ystore(p):
        pltpu.make_async_copy(ybuf2.at[p], yidx_hbm.at[pl.ds(0, CH)],
                              sem_sy[p]).wait()

    def wait_gather(p):
        pltpu.make_async_copy(winv_hbm.at[pl.ds(0, CH)], wbuf2.at[p],
                              sem_g[p]).wait()

    def do_sup(u, par0, early_pred):
        off = pl.multiple_of(base + u * WSUPE, 8)
        pltpu.sync_copy(src_hbm.at[pl.ds(off, WSUPE)], srcS)
        pltpu.sync_copy(dst_hbm.at[pl.ds(off, WSUPE)], dstS)
        pltpu.sync_copy(et_hbm.at[pl.ds(off, WSUPE)], etS)
        for k in range(WSUP):
            p = (k + par0) & 1
            q = 1 - p
            ebase = pl.multiple_of(off + k * CH, 8)

            def drains(p=p):
                drain_wstore(p)
                drain_ystore(p)

            if early_pred is not None and k < 2:
                pl.when(early_pred)(drains)
            else:
                drains()
            for m in range(CH // 16):
                sv = srcS[pl.ds(k * CH + m * 16, 16)]
                dv = dstS[pl.ds(k * CH + m * 16, 16)]
                ev = etS[pl.ds(k * CH + m * 16, 16)]
                g = ev * NN + dv + jnp.where(ev >= 4, WIDX_SHIFT, 0)
                idxr2[p, pl.ds(m * 16, 16)] = jnp.minimum(g, TRASH_W)
                ybuf2[p, pl.ds(m * 16, 16)] = ev * NN + sv
            pltpu.async_copy(ybuf2.at[p], yidx_hbm.at[pl.ds(ebase, CH)],
                             sem_sy[p])
            pltpu.async_copy(winv_hbm.at[idxr2.at[p]], wbuf2.at[p], sem_g[p])
            prev = pl.multiple_of(ebase - CH, 8)

            def finish_prev(q=q, prev=prev):
                wait_gather(q)
                pltpu.async_copy(wbuf2.at[q], w_hbm.at[pl.ds(prev, CH)],
                                 sem_sw[q])

            if early_pred is not None and k == 0:
                pl.when(early_pred)(finish_prev)
            else:
                finish_prev()

    def pair(j, _):
        do_sup(2 * j, 0, j > 0)
        do_sup(2 * j + 1, 1, None)
        return ()

    lax.fori_loop(0, WPAIRS, pair, ())
    do_sup(WNSUP - 1, 0, None)
    lastp = (WSUP - 1) & 1
    last_ebase = pl.multiple_of(base + span - CH, 8)
    wait_gather(lastp)
    pltpu.async_copy(wbuf2.at[lastp], w_hbm.at[pl.ds(last_ebase, CH)],
                     sem_sw[lastp])
    drain_wstore(0)
    drain_wstore(1)
    drain_ystore(0)
    drain_ystore(1)


ECH = 64
SUP = 7
SUPE = SUP * ECH
NSUPT = (E_PAD // NSUB) // SUPE
PAIRS = NSUPT // 2


@functools.partial(
    pl.kernel, mesh=_MESH, compiler_params=_SC_PARAMS,
    out_type=jax.ShapeDtypeStruct((NN, HH), jnp.float32),
    scratch_types=[
        pltpu.VMEM_SHARED((ACC_ROWS, HH), jnp.float32),
        pltpu.VMEM((SUPE,), jnp.int32),
        pltpu.VMEM((SUPE,), jnp.int32),
        pltpu.VMEM((SUPE + 16,), jnp.float32),
        pltpu.VMEM((2, ECH), jnp.int32),
        pltpu.VMEM((2, ECH, HH), jnp.float32),
        pltpu.SemaphoreType.DMA,
        pltpu.SemaphoreType.DMA,
        pltpu.SemaphoreType.DMA,
        pltpu.SemaphoreType.DMA,
        pltpu.SemaphoreType.DMA,
        pltpu.SemaphoreType.DMA,
        pltpu.SemaphoreType.DMA,
    ],
)
def _sc_edge(yflat_hbm, base_hbm, yidx_hbm, dst_hbm, w_hbm, out_hbm,
             acc, yicS, dstS, wS, ldst2, rowbuf2,
             sem_sy, sem_sd, sem_sw, sem_g0, sem_g1, sem_w0, sem_w1):
    c = lax.axis_index("c")
    s = lax.axis_index("s")
    lo = c * NHALF
    span = E_PAD // NSUB
    sem_g = (sem_g0, sem_g1)
    sem_w = (sem_w0, sem_w1)

    rows_pt = 1880
    last = NSUB - 1

    @pl.when(s < last)
    def _():
        pltpu.sync_copy(base_hbm.at[pl.ds(lo + s * rows_pt, rows_pt)],
                        acc.at[pl.ds(s * rows_pt, rows_pt)])

    @pl.when(s == last)
    def _():
        rem = NHALF - last * rows_pt
        pltpu.sync_copy(base_hbm.at[pl.ds(lo + last * rows_pt, rem)],
                        acc.at[pl.ds(last * rows_pt, rem)])
        for j in range(8):
            for q in range(HH // 16):
                rowbuf2[0, j, pl.ds(q * 16, 16)] = jnp.zeros((16,), jnp.float32)
        pltpu.sync_copy(rowbuf2.at[0, pl.ds(0, 8)], acc.at[pl.ds(NHALF, 8)])

    plsc.subcore_barrier()

    def start_sup(u):
        off = pl.multiple_of(s * span + u * SUPE, 8)
        pltpu.async_copy(yidx_hbm.at[pl.ds(off, SUPE)], yicS, sem_sy)
        pltpu.async_copy(dst_hbm.at[pl.ds(off, SUPE)], dstS, sem_sd)
        pltpu.async_copy(w_hbm.at[pl.ds(off, SUPE)], wS.at[pl.ds(0, SUPE)],
                         sem_sw)

    def wait_sup_yic():
        pltpu.make_async_copy(yidx_hbm.at[pl.ds(0, SUPE)], yicS, sem_sy).wait()

    def wait_sup_dw():
        pltpu.make_async_copy(dst_hbm.at[pl.ds(0, SUPE)], dstS, sem_sd).wait()
        pltpu.make_async_copy(w_hbm.at[pl.ds(0, SUPE)],
                              wS.at[pl.ds(0, SUPE)], sem_sw).wait()

    def start_gather(kk, p):
        pltpu.async_copy(yflat_hbm.at[yicS.at[pl.ds(kk * ECH, ECH)]],
                         rowbuf2.at[p], sem_g[p])

    def wait_gather(p):
        pltpu.make_async_copy(yflat_hbm.at[pl.ds(0, ECH)], rowbuf2.at[p],
                              sem_g[p]).wait()

    def start_scatter(p):
        pltpu.async_copy(rowbuf2.at[p], acc.at[ldst2.at[p]], sem_w[p],
                         add=True)

    def wait_scatter(p):
        pltpu.make_async_copy(yflat_hbm.at[pl.ds(0, ECH)], rowbuf2.at[p],
                              sem_w[p]).wait()

    def compute_chunk(kk, p):
        for m in range(ECH // 16):
            dv = dstS[pl.ds(kk * ECH + m * 16, 16)]
            l = dv - lo
            own = (l >= 0) & (l < NHALF)
            ldst2[p, pl.ds(m * 16, 16)] = jnp.where(
                own, l, NHALF + jnp.bitwise_and(_IOTA16(), 7))

        def scale_body(e, _):
            wv = wS[pl.ds(kk * ECH + e, 16)]
            we = wv[0]
            for q in range(HH // 16):
                rowbuf2[p, e, pl.ds(q * 16, 16)] = (
                    rowbuf2[p, e, pl.ds(q * 16, 16)] * jnp.full((16,), we))
            return ()

        lax.fori_loop(0, ECH, scale_body, (), unroll=2)

    start_sup(0)
    wait_sup_yic()
    wait_sup_dw()
    start_gather(0, 0)

    def pair(j, _):
        for k in range(2 * SUP):
            kk = k % SUP
            p = k & 1
            q = 1 - p
            wait_gather(p)
            if k == 0:
                @pl.when(j > 0)
                def _():
                    wait_sup_dw()
            if k == SUP:
                wait_sup_dw()
            if kk != SUP - 1:
                if k == 0:
                    @pl.when(j > 0)
                    def _():
                        wait_scatter(q)
                else:
                    wait_scatter(q)
                start_gather(kk + 1, q)
                compute_chunk(kk, p)
            else:
                compute_chunk(kk, p)
                islast = (k == 2 * SUP - 1) & (j == PAIRS - 1)
                if k == 2 * SUP - 1:
                    @pl.when(j < PAIRS - 1)
                    def _():
                        start_sup(2 * j + 2)
                        wait_sup_yic()
                        wait_scatter(q)
                        start_gather(0, q)
                else:
                    start_sup(2 * j + 1)
                    wait_sup_yic()
                    wait_scatter(q)
                    start_gather(0, q)
            start_scatter(p)
        return ()

    lax.fori_loop(0, PAIRS, pair, ())
    wait_scatter(0)
    wait_scatter(1)
    plsc.subcore_barrier()

    @pl.when(s < last)
    def _():
        pltpu.sync_copy(acc.at[pl.ds(s * rows_pt, rows_pt)],
                        out_hbm.at[pl.ds(lo + s * rows_pt, rows_pt)])

    @pl.when(s == last)
    def _():
        rem = NHALF - last * rows_pt
        pltpu.sync_copy(acc.at[pl.ds(last * rows_pt, rem)],
                        out_hbm.at[pl.ds(lo + last * rows_pt, rem)])


def kernel(x_transaction, edge_index, edge_type, W_in, b_in, emb_account,
           emb_merchant, W_rel0, W_root0, b0, g0, beta0, W_rel1, W_root1, b1,
           g1, beta1, Wc1, bc1, gc, betac, Wc2, bc2):
    xt = _input_proj(x_transaction, W_in, b_in)
    x = jnp.concatenate([xt, emb_account, emb_merchant], axis=0)

    pad = E_PAD - EE
    src = jnp.concatenate([edge_index[0], jnp.zeros((pad,), jnp.int32)])
    dst = jnp.concatenate([edge_index[1], jnp.full((pad,), 1 << 29, jnp.int32)])
    et = jnp.concatenate([edge_type, jnp.zeros((pad,), jnp.int32)])

    winv = _sc_deg(dst, et)
    w, yidx = _sc_wsweep(src, dst, et, winv)

    y0, base0 = _transforms(x, W_rel0, W_root0, b0)
    x = _post(_sc_edge(y0, base0, yidx, dst, w), g0, beta0)
    y1, base1 = _transforms(x, W_rel1, W_root1, b1)
    x = _post(_sc_edge(y1, base1, yidx, dst, w), g1, beta1)

    return _classifier(x[:N_T_], Wc1, bc1, gc, betac, Wc2, bc2)


# speedup vs baseline: 13.8692x; 1.1146x over previous
import functools

import jax
import jax.numpy as jnp
from jax import lax
from jax.experimental import pallas as pl
from jax.experimental.pallas import tpu as pltpu
from jax.experimental.pallas import tpu_sc as plsc

N_T_, N_A_, N_M_ = 50000, 5000, 5000
NN = N_T_ + N_A_ + N_M_
EE = 600000
DIN, HH, NREL = 128, 64, 7

E_PAD = 602112
CH = 128
NSUB = 16
NCORE = 2

ACC_D = 241664
DEG_SPAN = 15104
DEG_SUB = 944
WINV_SZ = 2 * ACC_D
WIDX_SHIFT = ACC_D - 4 * NN
TRASH_W = ACC_D + 3 * NN

WSUP = 7
WSUPE = WSUP * CH
WNSUP = (E_PAD // (NSUB * NCORE)) // WSUPE
WPAIRS = (WNSUP - 1) // 2

NHALF = NN // 2
ACC_ROWS = NHALF + 8


def _nan_fix(x, pos, neg):
    y = jnp.where(jnp.isnan(x), 0.0, x)
    y = jnp.where(x == jnp.inf, pos, y)
    y = jnp.where(x == -jnp.inf, neg, y)
    return y


def _proj_body(x_ref, w_ref, b_ref, o_ref):
    xb = jnp.clip(x_ref[...], -10.0, 10.0)
    o_ref[...] = jnp.dot(xb, w_ref[...], preferred_element_type=jnp.float32) + b_ref[...]


def _input_proj(xt, w, b):
    blk = 2000
    return pl.pallas_call(
        _proj_body,
        grid=(N_T_ // blk,),
        in_specs=[
            pl.BlockSpec((blk, DIN), lambda i: (i, 0)),
            pl.BlockSpec((DIN, HH), lambda i: (0, 0)),
            pl.BlockSpec((1, HH), lambda i: (0, 0)),
        ],
        out_specs=pl.BlockSpec((blk, HH), lambda i: (i, 0)),
        out_shape=jax.ShapeDtypeStruct((N_T_, HH), jnp.float32),
    )(xt, w, b.reshape(1, HH))


def _transforms_body(x_ref, wrel_ref, wroot_ref, b_ref, y_ref, base_ref):
    xb = x_ref[...]
    for r in range(NREL):
        y_ref[r] = jnp.dot(xb, wrel_ref[r], preferred_element_type=jnp.float32)
    base_ref[...] = jnp.dot(xb, wroot_ref[...], preferred_element_type=jnp.float32) + b_ref[...]


def _transforms(x, w_rel, w_root, b):
    blk = 2000
    y, base = pl.pallas_call(
        _transforms_body,
        grid=(NN // blk,),
        in_specs=[
            pl.BlockSpec((blk, HH), lambda i: (i, 0)),
            pl.BlockSpec((NREL, HH, HH), lambda i: (0, 0, 0)),
            pl.BlockSpec((HH, HH), lambda i: (0, 0)),
            pl.BlockSpec((1, HH), lambda i: (0, 0)),
        ],
        out_specs=[
            pl.BlockSpec((NREL, blk, HH), lambda i: (0, i, 0)),
            pl.BlockSpec((blk, HH), lambda i: (i, 0)),
        ],
        out_shape=[
            jax.ShapeDtypeStruct((NREL, NN, HH), jnp.float32),
            jax.ShapeDtypeStruct((NN, HH), jnp.float32),
        ],
    )(x, w_rel, w_root, b.reshape(1, HH))
    return y.reshape(NREL * NN, HH), base


def _post_body(x_ref, g_ref, b_ref, o_ref):
    x = x_ref[...]
    mu = jnp.mean(x, axis=1, keepdims=True)
    var = jnp.mean((x - mu) ** 2, axis=1, keepdims=True)
    y = (x - mu) / jnp.sqrt(var + 1e-5) * g_ref[...] + b_ref[...]
    y = jnp.maximum(y, 0.0)
    o_ref[...] = _nan_fix(y, 1.0, -1.0)


def _post(x, g, beta):
    blk = 2000
    return pl.pallas_call(
        _post_body,
        grid=(NN // blk,),
        in_specs=[
            pl.BlockSpec((blk, HH), lambda i: (i, 0)),
            pl.BlockSpec((1, HH), lambda i: (0, 0)),
            pl.BlockSpec((1, HH), lambda i: (0, 0)),
        ],
        out_specs=pl.BlockSpec((blk, HH), lambda i: (i, 0)),
        out_shape=jax.ShapeDtypeStruct((NN, HH), jnp.float32),
    )(x, g.reshape(1, HH), beta.reshape(1, HH))


def _clf_body(t_ref, w1_ref, b1_ref, g_ref, bt_ref, w2_ref, b2_ref, o_ref):
    h = jnp.dot(t_ref[...], w1_ref[...], preferred_element_type=jnp.float32) + b1_ref[...]
    mu = jnp.mean(h, axis=1, keepdims=True)
    var = jnp.mean((h - mu) ** 2, axis=1, keepdims=True)
    h = (h - mu) / jnp.sqrt(var + 1e-5) * g_ref[...] + bt_ref[...]
    h = jnp.maximum(h, 0.0)
    lo = jnp.dot(h, w2_ref[...], preferred_element_type=jnp.float32) + b2_ref[...]
    o_ref[...] = _nan_fix(lo, 5.0, -5.0)


def _classifier(t, w1, b1, g, beta, w2, b2):
    blk = 2000
    hh2 = HH // 2
    return pl.pallas_call(
        _clf_body,
        grid=(N_T_ // blk,),
        in_specs=[
            pl.BlockSpec((blk, HH), lambda i: (i, 0)),
            pl.BlockSpec((HH, hh2), lambda i: (0, 0)),
            pl.BlockSpec((1, hh2), lambda i: (0, 0)),
            pl.BlockSpec((1, hh2), lambda i: (0, 0)),
            pl.BlockSpec((1, hh2), lambda i: (0, 0)),
            pl.BlockSpec((hh2, 1), lambda i: (0, 0)),
            pl.BlockSpec((1, 1), lambda i: (0, 0)),
        ],
        out_specs=pl.BlockSpec((blk, 1), lambda i: (i, 0)),
        out_shape=jax.ShapeDtypeStruct((N_T_, 1), jnp.float32),
    )(t, w1, b1.reshape(1, hh2), g.reshape(1, hh2), beta.reshape(1, hh2),
      w2, b2.reshape(1, 1))


_MESH = plsc.VectorSubcoreMesh(core_axis_name="c", subcore_axis_name="s")
_SC_PARAMS = pltpu.CompilerParams(use_tc_tiling_on_sc=False)
_IOTA16 = lambda: lax.iota(jnp.int32, 16)


@functools.partial(
    pl.kernel, mesh=_MESH, compiler_params=_SC_PARAMS,
    out_type=jax.ShapeDtypeStruct((WINV_SZ,), jnp.float32),
    scratch_types=[
        pltpu.VMEM_SHARED((ACC_D,), jnp.float32),
        pltpu.VMEM((DEG_SUB,), jnp.float32),
        pltpu.VMEM((WSUPE,), jnp.int32),
        pltpu.VMEM((WSUPE,), jnp.int32),
        pltpu.VMEM((2, CH), jnp.int32),
        pltpu.VMEM((CH,), jnp.float32),
        pltpu.VMEM((16,), jnp.float32),
        pltpu.SemaphoreType.DMA,
        pltpu.SemaphoreType.DMA,
    ],
)
def _sc_deg(dst_hbm, et_hbm, winv_hbm, acc, tbuf, dstS, etS, idxr2, onesb, z16,
            sem_h0, sem_h1):
    c = lax.axis_index("c")
    s = lax.axis_index("s")
    nrel = 4 - c
    rbase = 4 * c
    trash = 60000 * nrel

    for j in range(DEG_SUB // 16):
        tbuf[pl.ds(j * 16, 16)] = jnp.zeros((16,), jnp.float32)
    for j in range(CH // 16):
        onesb[pl.ds(j * 16, 16)] = jnp.ones((16,), jnp.float32)
    z16[...] = jnp.zeros((16,), jnp.float32)
    for i in range(NSUB):
        pltpu.sync_copy(tbuf, acc.at[pl.ds(s * DEG_SPAN + i * DEG_SUB, DEG_SUB)])
    plsc.subcore_barrier()

    span = E_PAD // NSUB
    sem_h = (sem_h0, sem_h1)

    def drain_hist(p):
        pltpu.make_async_copy(et_hbm.at[pl.ds(0, CH)], idxr2.at[p],
                              sem_h[p]).wait()

    def do_sup(u, par0, early_pred):
        off = pl.multiple_of(s * span + u * WSUPE, 8)
        pltpu.sync_copy(dst_hbm.at[pl.ds(off, WSUPE)], dstS)
        pltpu.sync_copy(et_hbm.at[pl.ds(off, WSUPE)], etS)
        for k in range(WSUP):
            p = (k + par0) & 1
            if early_pred is not None and k < 2:
                pl.when(early_pred)(lambda p=p: drain_hist(p))
            else:
                drain_hist(p)
            for m in range(CH // 16):
                dv = dstS[pl.ds(k * CH + m * 16, 16)]
                ev = etS[pl.ds(k * CH + m * 16, 16)]
                rel = ev - rbase
                own = (rel >= 0) & (rel < nrel) & (dv >= 0) & (dv < NN)
                hx = jnp.where(own, rel * NN + dv, trash + _IOTA16())
                idxr2[p, pl.ds(m * 16, 16)] = hx
            pltpu.async_copy(onesb, acc.at[idxr2.at[p]], sem_h[p], add=True)

    def pair(j, _):
        do_sup(2 * j, 0, j > 0)
        do_sup(2 * j + 1, 1, None)
        return ()

    lax.fori_loop(0, (span // WSUPE) // 2, pair, ())
    drain_hist(0)
    drain_hist(1)
    plsc.subcore_barrier()

    def winv_chunk(i, _):
        off = s * DEG_SPAN + i * DEG_SUB
        pltpu.sync_copy(acc.at[pl.ds(off, DEG_SUB)], tbuf)
        for j in range(DEG_SUB // 16):
            v = tbuf[pl.ds(j * 16, 16)]
            tbuf[pl.ds(j * 16, 16)] = 1.0 / jnp.maximum(v, 1.0)
        pltpu.sync_copy(tbuf, winv_hbm.at[pl.ds(c * ACC_D + off, DEG_SUB)])
        return ()

    lax.fori_loop(0, NSUB, winv_chunk, ())
    plsc.subcore_barrier()

    @pl.when((c == 1) & (s == 0))
    def _():
        pltpu.sync_copy(z16, winv_hbm.at[pl.ds(TRASH_W, 16)])


@functools.partial(
    pl.kernel, mesh=_MESH, compiler_params=_SC_PARAMS,
    out_type=[
        jax.ShapeDtypeStruct((E_PAD,), jnp.float32),
        jax.ShapeDtypeStruct((E_PAD,), jnp.int32),
    ],
    scratch_types=[
        pltpu.VMEM((WSUPE,), jnp.int32),
        pltpu.VMEM((WSUPE,), jnp.int32),
        pltpu.VMEM((WSUPE,), jnp.int32),
        pltpu.VMEM((2, CH), jnp.int32),
        pltpu.VMEM((2, CH), jnp.int32),
        pltpu.VMEM((2, CH), jnp.float32),
        pltpu.SemaphoreType.DMA,
        pltpu.SemaphoreType.DMA,
        pltpu.SemaphoreType.DMA,
        pltpu.SemaphoreType.DMA,
        pltpu.SemaphoreType.DMA,
        pltpu.SemaphoreType.DMA,
    ],
)
def _sc_wsweep(src_hbm, dst_hbm, et_hbm, winv_hbm, w_hbm, yidx_hbm,
               srcS, dstS, etS, idxr2, ybuf2, wbuf2,
               sem_g0, sem_g1, sem_sw0, sem_sw1, sem_sy0, sem_sy1):
    c = lax.axis_index("c")
    s = lax.axis_index("s")
    wid = s * NCORE + c
    span = E_PAD // (NSUB * NCORE)
    base = wid * span
    sem_g = (sem_g0, sem_g1)
    sem_sw = (sem_sw0, sem_sw1)
    sem_sy = (sem_sy0, sem_sy1)

    def drain_wstore(p):
        pltpu.make_async_copy(wbuf2.at[p], w_hbm.at[pl.ds(0, CH)],
                              sem_sw[p]).wait()

    def drain_ystore(p):
        pltpu.make_async_copy(ybuf2.at[p], yidx_hbm.at[pl.ds(0, CH)],
                              sem_sy[p]).wait()

    def wait_gather(p):
        pltpu.make_async_copy(winv_hbm.at[pl.ds(0, CH)], wbuf2.at[p],
                              sem_g[p]).wait()

    def do_sup(u, par0, early_pred):
        off = pl.multiple_of(base + u * WSUPE, 8)
        pltpu.sync_copy(src_hbm.at[pl.ds(off, WSUPE)], srcS)
        pltpu.sync_copy(dst_hbm.at[pl.ds(off, WSUPE)], dstS)
        pltpu.sync_copy(et_hbm.at[pl.ds(off, WSUPE)], etS)
        for k in range(WSUP):
            p = (k + par0) & 1
            q = 1 - p
            ebase = pl.multiple_of(off + k * CH, 8)

            def drains(p=p):
                drain_wstore(p)
                drain_ystore(p)

            if early_pred is not None and k < 2:
                pl.when(early_pred)(drains)
            else:
                drains()
            for m in range(CH // 16):
                sv = srcS[pl.ds(k * CH + m * 16, 16)]
                dv = dstS[pl.ds(k * CH + m * 16, 16)]
                ev = etS[pl.ds(k * CH + m * 16, 16)]
                g = ev * NN + dv + jnp.where(ev >= 4, WIDX_SHIFT, 0)
                idxr2[p, pl.ds(m * 16, 16)] = jnp.minimum(g, TRASH_W)
                ybuf2[p, pl.ds(m * 16, 16)] = ev * NN + sv
            pltpu.async_copy(ybuf2.at[p], yidx_hbm.at[pl.ds(ebase, CH)],
                             sem_sy[p])
            pltpu.async_copy(winv_hbm.at[idxr2.at[p]], wbuf2.at[p], sem_g[p])
            prev = pl.multiple_of(ebase - CH, 8)

            def finish_prev(q=q, prev=prev):
                wait_gather(q)
                pltpu.async_copy(wbuf2.at[q], w_hbm.at[pl.ds(prev, CH)],
                                 sem_sw[q])

            if early_pred is not None and k == 0:
                pl.when(early_pred)(finish_prev)
            else:
                finish_prev()

    def pair(j, _):
        do_sup(2 * j, 0, j > 0)
        do_sup(2 * j + 1, 1, None)
        return ()

    lax.fori_loop(0, WPAIRS, pair, ())
    do_sup(WNSUP - 1, 0, None)
    lastp = (WSUP - 1) & 1
    last_ebase = pl.multiple_of(base + span - CH, 8)
    wait_gather(lastp)
    pltpu.async_copy(wbuf2.at[lastp], w_hbm.at[pl.ds(last_ebase, CH)],
                     sem_sw[lastp])
    drain_wstore(0)
    drain_wstore(1)
    drain_ystore(0)
    drain_ystore(1)


ECH = 64
SUP = 7
SUPE = SUP * ECH
NSUPT = (E_PAD // NSUB) // SUPE
PAIRS = NSUPT // 2


@functools.partial(
    pl.kernel, mesh=_MESH, compiler_params=_SC_PARAMS,
    out_type=jax.ShapeDtypeStruct((NN, HH), jnp.float32),
    scratch_types=[
        pltpu.VMEM_SHARED((ACC_ROWS, HH), jnp.float32),
        pltpu.VMEM((SUPE,), jnp.int32),
        pltpu.VMEM((SUPE,), jnp.int32),
        pltpu.VMEM((SUPE + 16,), jnp.float32),
        pltpu.VMEM((2, ECH), jnp.int32),
        pltpu.VMEM((2, ECH, HH), jnp.float32),
        pltpu.SemaphoreType.DMA,
        pltpu.SemaphoreType.DMA,
        pltpu.SemaphoreType.DMA,
        pltpu.SemaphoreType.DMA,
        pltpu.SemaphoreType.DMA,
        pltpu.SemaphoreType.DMA,
        pltpu.SemaphoreType.DMA,
    ],
)
def _sc_edge(yflat_hbm, base_hbm, yidx_hbm, dst_hbm, w_hbm, out_hbm,
             acc, yicS, dstS, wS, ldst2, rowbuf2,
             sem_sy, sem_sd, sem_sw, sem_g0, sem_g1, sem_w0, sem_w1):
    c = lax.axis_index("c")
    s = lax.axis_index("s")
    lo = c * NHALF
    span = E_PAD // NSUB
    sem_g = (sem_g0, sem_g1)
    sem_w = (sem_w0, sem_w1)

    rows_pt = 1880
    last = NSUB - 1

    @pl.when(s < last)
    def _():
        pltpu.sync_copy(base_hbm.at[pl.ds(lo + s * rows_pt, rows_pt)],
                        acc.at[pl.ds(s * rows_pt, rows_pt)])

    @pl.when(s == last)
    def _():
        rem = NHALF - last * rows_pt
        pltpu.sync_copy(base_hbm.at[pl.ds(lo + last * rows_pt, rem)],
                        acc.at[pl.ds(last * rows_pt, rem)])
        for j in range(8):
            for q in range(HH // 16):
                rowbuf2[0, j, pl.ds(q * 16, 16)] = jnp.zeros((16,), jnp.float32)
        pltpu.sync_copy(rowbuf2.at[0, pl.ds(0, 8)], acc.at[pl.ds(NHALF, 8)])

    plsc.subcore_barrier()

    def start_sup(u):
        off = pl.multiple_of(s * span + u * SUPE, 8)
        pltpu.async_copy(yidx_hbm.at[pl.ds(off, SUPE)], yicS, sem_sy)
        pltpu.async_copy(dst_hbm.at[pl.ds(off, SUPE)], dstS, sem_sd)
        pltpu.async_copy(w_hbm.at[pl.ds(off, SUPE)], wS.at[pl.ds(0, SUPE)],
                         sem_sw)

    def wait_sup_yic():
        pltpu.make_async_copy(yidx_hbm.at[pl.ds(0, SUPE)], yicS, sem_sy).wait()

    def wait_sup_dw():
        pltpu.make_async_copy(dst_hbm.at[pl.ds(0, SUPE)], dstS, sem_sd).wait()
        pltpu.make_async_copy(w_hbm.at[pl.ds(0, SUPE)],
                              wS.at[pl.ds(0, SUPE)], sem_sw).wait()

    def start_gather(kk, p):
        pltpu.async_copy(yflat_hbm.at[yicS.at[pl.ds(kk * ECH, ECH)]],
                         rowbuf2.at[p], sem_g[p])

    def wait_gather(p):
        pltpu.make_async_copy(yflat_hbm.at[pl.ds(0, ECH)], rowbuf2.at[p],
                              sem_g[p]).wait()

    def start_scatter(p):
        pltpu.async_copy(rowbuf2.at[p], acc.at[ldst2.at[p]], sem_w[p],
                         add=True)

    def wait_scatter(p):
        pltpu.make_async_copy(yflat_hbm.at[pl.ds(0, ECH)], rowbuf2.at[p],
                              sem_w[p]).wait()

    def compute_chunk(kk, p):
        for m in range(ECH // 16):
            dv = dstS[pl.ds(kk * ECH + m * 16, 16)]
            l = dv - lo
            own = (l >= 0) & (l < NHALF)
            ldst2[p, pl.ds(m * 16, 16)] = jnp.where(
                own, l, NHALF + jnp.bitwise_and(_IOTA16(), 7))

        def scale_body(e, _):
            wv = wS[pl.ds(kk * ECH + e, 16)]
            we = wv[0]
            for q in range(HH // 16):
                rowbuf2[p, e, pl.ds(q * 16, 16)] = (
                    rowbuf2[p, e, pl.ds(q * 16, 16)] * jnp.full((16,), we))
            return ()

        lax.fori_loop(0, ECH, scale_body, (), unroll=2)

    start_sup(0)
    wait_sup_yic()
    wait_sup_dw()
    start_gather(0, 0)

    def pair(j, _):
        for k in range(2 * SUP):
            kk = k % SUP
            p = k & 1
            q = 1 - p
            wait_gather(p)
            if k == 0:
                @pl.when(j > 0)
                def _():
                    wait_sup_dw()
            if k == SUP:
                wait_sup_dw()
            if kk != SUP - 1:
                if k == 0:
                    @pl.when(j > 0)
                    def _():
                        wait_scatter(q)
                else:
                    wait_scatter(q)
                start_gather(kk + 1, q)
                compute_chunk(kk, p)
            else:
                compute_chunk(kk, p)
                islast = (k == 2 * SUP - 1) & (j == PAIRS - 1)
                if k == 2 * SUP - 1:
                    @pl.when(j < PAIRS - 1)
                    def _():
                        start_sup(2 * j + 2)
                        wait_sup_yic()
                        wait_scatter(q)
                        start_gather(0, q)
                else:
                    start_sup(2 * j + 1)
                    wait_sup_yic()
                    wait_scatter(q)
                    start_gather(0, q)
            start_scatter(p)
        return ()

    lax.fori_loop(0, PAIRS, pair, ())
    wait_scatter(0)
    wait_scatter(1)
    plsc.subcore_barrier()

    @pl.when(s < last)
    def _():
        pltpu.sync_copy(acc.at[pl.ds(s * rows_pt, rows_pt)],
                        out_hbm.at[pl.ds(lo + s * rows_pt, rows_pt)])

    @pl.when(s == last)
    def _():
        rem = NHALF - last * rows_pt
        pltpu.sync_copy(acc.at[pl.ds(last * rows_pt, rem)],
                        out_hbm.at[pl.ds(lo + last * rows_pt, rem)])


def kernel(x_transaction, edge_index, edge_type, W_in, b_in, emb_account,
           emb_merchant, W_rel0, W_root0, b0, g0, beta0, W_rel1, W_root1, b1,
           g1, beta1, Wc1, bc1, gc, betac, Wc2, bc2):
    xt = _input_proj(x_transaction, W_in, b_in)
    x = jnp.concatenate([xt, emb_account, emb_merchant], axis=0)

    pad = E_PAD - EE
    src = jnp.concatenate([edge_index[0], jnp.zeros((pad,), jnp.int32)])
    dst = jnp.concatenate([edge_index[1], jnp.full((pad,), 1 << 29, jnp.int32)])
    et = jnp.concatenate([edge_type, jnp.zeros((pad,), jnp.int32)])

    winv = _sc_deg(dst, et)
    w, yidx = _sc_wsweep(src, dst, et, winv)

    y0, base0 = _transforms(x, W_rel0, W_root0, b0)
    x = _post(_sc_edge(y0, base0, yidx, dst, w), g0, beta0)
    y1, base1 = _transforms(x, W_rel1, W_root1, b1)
    x = _post(_sc_edge(y1, base1, yidx, dst, w), g1, beta1)

    return _classifier(x[:N_T_], Wc1, bc1, gc, betac, Wc2, bc2)
